# Initial kernel scaffold; baseline (speedup 1.0000x reference)
import jax
import jax.numpy as jnp
from jax.experimental import pallas as pl


def kernel(z, edge_index):
    raise NotImplementedError("write your pallas kernel here")


# speedup vs baseline: 3.4983x; 3.4983x over previous
import functools

import jax
import jax.numpy as jnp
from jax import lax
from jax.experimental import pallas as pl
from jax.experimental.pallas import tpu as pltpu
from jax.experimental.pallas import tpu_sc as plsc

D = 128
CH = 128
L = 16
NW = 32


def _decoder(z_hbm, src_hbm, dst_hbm, out_hbm,
             idx_s, idx_d, rows_s, rows_d, out_v, t16, sem_s, sem_d):
    num_chunks = src_hbm.shape[0] // CH
    wid = lax.axis_index("s") * 2 + lax.axis_index("c")
    c_lo = (wid * num_chunks) // NW
    c_hi = ((wid + 1) * num_chunks) // NW

    lane = lax.iota(jnp.int32, L)

    def chunk_body(c, _):
        base = c * CH
        pltpu.sync_copy(src_hbm.at[pl.ds(base, CH)], idx_s)
        pltpu.sync_copy(dst_hbm.at[pl.ds(base, CH)], idx_d)
        cp_s = pltpu.async_copy(z_hbm.at[idx_s], rows_s, sem_s)
        cp_d = pltpu.async_copy(z_hbm.at[idx_d], rows_d, sem_d)
        cp_s.wait()
        cp_d.wait()

        def sub_body(sb, _):
            e0 = sb * L
            for i in range(L):
                e = e0 + i
                acc = rows_s[e, pl.ds(0, L)] * rows_d[e, pl.ds(0, L)]
                for j in range(1, D // L):
                    acc += (rows_s[e, pl.ds(j * L, L)]
                            * rows_d[e, pl.ds(j * L, L)])
                plsc.store_scatter(t16, [lane, jnp.full((L,), i, jnp.int32)],
                                   acc)
            s = t16[0, pl.ds(0, L)]
            for r in range(1, L):
                s += t16[r, pl.ds(0, L)]
            out_v[pl.ds(e0, L)] = 1.0 / (1.0 + jnp.exp(-s))
            return ()

        lax.fori_loop(0, CH // L, sub_body, ())
        pltpu.sync_copy(out_v, out_hbm.at[pl.ds(base, CH)])
        return ()

    lax.fori_loop(c_lo, c_hi, chunk_body, ())


def kernel(z, edge_index):
    e = edge_index.astype(jnp.int32)
    src, dst = e[0], e[1]
    n_edges = src.shape[0]
    run = functools.partial(
        pl.kernel,
        out_type=jax.ShapeDtypeStruct((n_edges,), jnp.float32),
        mesh=plsc.VectorSubcoreMesh(core_axis_name="c", subcore_axis_name="s"),
        compiler_params=pltpu.CompilerParams(needs_layout_passes=False),
        scratch_types=[
            pltpu.VMEM((CH,), jnp.int32),
            pltpu.VMEM((CH,), jnp.int32),
            pltpu.VMEM((CH, D), jnp.float32),
            pltpu.VMEM((CH, D), jnp.float32),
            pltpu.VMEM((CH,), jnp.float32),
            pltpu.VMEM((L, L), jnp.float32),
            pltpu.SemaphoreType.DMA,
            pltpu.SemaphoreType.DMA,
        ],
    )(_decoder)
    return run(z, src, dst)


# speedup vs baseline: 6.1812x; 1.7669x over previous
import functools

import jax
import jax.numpy as jnp
from jax import lax
from jax.experimental import pallas as pl
from jax.experimental.pallas import tpu as pltpu
from jax.experimental.pallas import tpu_sc as plsc

D = 128
CH = 128
L = 16
NW = 32
NCHUNK = 80


def _decoder(z_hbm, src_hbm, dst_hbm, out_hbm,
             idx_s0, idx_s1, idx_d0, idx_d1,
             rows_s0, rows_s1, rows_d0, rows_d1,
             out_v, t16,
             sem_i0, sem_i1, sem_r0, sem_r1):
    idx_s = [idx_s0, idx_s1]
    idx_d = [idx_d0, idx_d1]
    rows_s = [rows_s0, rows_s1]
    rows_d = [rows_d0, rows_d1]
    sem_i = [sem_i0, sem_i1]
    sem_r = [sem_r0, sem_r1]

    edges_per_w = src_hbm.shape[0] // NW
    last_base = edges_per_w - CH
    wid = lax.axis_index("s") * 2 + lax.axis_index("c")
    w0 = wid * edges_per_w

    lane = lax.iota(jnp.int32, L)

    def base_of(c):
        return w0 + jnp.minimum(c * CH, last_base)

    def issue_idx(c, b):
        base = base_of(c)
        pltpu.async_copy(src_hbm.at[pl.ds(base, CH)], idx_s[b], sem_i[b])
        pltpu.async_copy(dst_hbm.at[pl.ds(base, CH)], idx_d[b], sem_i[b])

    def wait_idx(b):
        pltpu.make_async_copy(src_hbm.at[pl.ds(0, CH)], idx_s[b],
                              sem_i[b]).wait()
        pltpu.make_async_copy(dst_hbm.at[pl.ds(0, CH)], idx_d[b],
                              sem_i[b]).wait()

    def issue_rows(b):
        pltpu.async_copy(z_hbm.at[idx_s[b]], rows_s[b], sem_r[b])
        pltpu.async_copy(z_hbm.at[idx_d[b]], rows_d[b], sem_r[b])

    def wait_rows(b):
        pltpu.make_async_copy(z_hbm.at[idx_s[b]], rows_s[b], sem_r[b]).wait()
        pltpu.make_async_copy(z_hbm.at[idx_d[b]], rows_d[b], sem_r[b]).wait()

    def compute_and_store(c, b):
        rs, rd = rows_s[b], rows_d[b]

        def sub_body(sb, _):
            e0 = sb * L
            for i in range(L):
                e = e0 + i
                acc = rs[e, pl.ds(0, L)] * rd[e, pl.ds(0, L)]
                for j in range(1, D // L):
                    acc += rs[e, pl.ds(j * L, L)] * rd[e, pl.ds(j * L, L)]
                plsc.store_scatter(t16, [lane, jnp.full((L,), i, jnp.int32)],
                                   acc)
            s = t16[0, pl.ds(0, L)]
            for r in range(1, L):
                s += t16[r, pl.ds(0, L)]
            out_v[pl.ds(e0, L)] = 1.0 / (1.0 + jnp.exp(-s))
            return ()

        lax.fori_loop(0, CH // L, sub_body, ())
        pltpu.sync_copy(out_v, out_hbm.at[pl.ds(base_of(c), CH)])

    issue_idx(0, 0)
    issue_idx(1, 1)
    wait_idx(0)
    issue_rows(0)

    def pair_body(g, _):
        for b in range(2):
            c = 2 * g + b
            nb = 1 - b
            wait_idx(nb)
            issue_rows(nb)
            wait_rows(b)
            issue_idx(c + 2, b)
            compute_and_store(c, b)
        return ()

    lax.fori_loop(0, (NCHUNK - 2) // 2, pair_body, ())

    wait_idx(1)
    issue_rows(1)
    wait_rows(0)
    compute_and_store(NCHUNK - 2, 0)
    wait_rows(1)
    compute_and_store(NCHUNK - 1, 1)


def kernel(z, edge_index):
    e = edge_index.astype(jnp.int32)
    src, dst = e[0], e[1]
    n_edges = src.shape[0]
    assert n_edges % NW == 0
    assert (n_edges // NW - CH) % 8 == 0
    assert (n_edges // NW) <= NCHUNK * CH
    run = functools.partial(
        pl.kernel,
        out_type=jax.ShapeDtypeStruct((n_edges,), jnp.float32),
        mesh=plsc.VectorSubcoreMesh(core_axis_name="c", subcore_axis_name="s"),
        compiler_params=pltpu.CompilerParams(needs_layout_passes=False),
        scratch_types=[
            pltpu.VMEM((CH,), jnp.int32),
            pltpu.VMEM((CH,), jnp.int32),
            pltpu.VMEM((CH,), jnp.int32),
            pltpu.VMEM((CH,), jnp.int32),
            pltpu.VMEM((CH, D), jnp.float32),
            pltpu.VMEM((CH, D), jnp.float32),
            pltpu.VMEM((CH, D), jnp.float32),
            pltpu.VMEM((CH, D), jnp.float32),
            pltpu.VMEM((CH,), jnp.float32),
            pltpu.VMEM((L, L), jnp.float32),
            pltpu.SemaphoreType.DMA,
            pltpu.SemaphoreType.DMA,
            pltpu.SemaphoreType.DMA,
            pltpu.SemaphoreType.DMA,
        ],
    )(_decoder)
    return run(z, src, dst)


# speedup vs baseline: 7.0757x; 1.1447x over previous
import functools

import jax
import jax.numpy as jnp
from jax import lax
from jax.experimental import pallas as pl
from jax.experimental.pallas import tpu as pltpu
from jax.experimental.pallas import tpu_sc as plsc

D = 128
DP = 64
CH = 128
L = 16
NW = 32
NB = 3
NCHUNK = 81


def _decoder(z_hbm, src_hbm, dst_hbm, out_hbm,
             idx_s0, idx_s1, idx_s2, idx_d0, idx_d1, idx_d2,
             rows_s0, rows_s1, rows_s2, rows_d0, rows_d1, rows_d2,
             out_v, t16,
             sem_i0, sem_i1, sem_i2, sem_r0, sem_r1, sem_r2):
    idx_s = [idx_s0, idx_s1, idx_s2]
    idx_d = [idx_d0, idx_d1, idx_d2]
    rows_s = [rows_s0, rows_s1, rows_s2]
    rows_d = [rows_d0, rows_d1, rows_d2]
    sem_i = [sem_i0, sem_i1, sem_i2]
    sem_r = [sem_r0, sem_r1, sem_r2]

    edges_per_w = src_hbm.shape[0] // NW
    last_base = edges_per_w - CH
    wid = lax.axis_index("s") * 2 + lax.axis_index("c")
    w0 = wid * edges_per_w

    lane = lax.iota(jnp.int32, L)

    def base_of(c):
        return w0 + jnp.minimum(c * CH, last_base)

    def issue_idx(c, b):
        base = base_of(c)
        pltpu.async_copy(src_hbm.at[pl.ds(base, CH)], idx_s[b], sem_i[b])
        pltpu.async_copy(dst_hbm.at[pl.ds(base, CH)], idx_d[b], sem_i[b])

    def wait_idx(b):
        pltpu.make_async_copy(src_hbm.at[pl.ds(0, CH)], idx_s[b],
                              sem_i[b]).wait()
        pltpu.make_async_copy(dst_hbm.at[pl.ds(0, CH)], idx_d[b],
                              sem_i[b]).wait()

    def issue_rows(b):
        pltpu.async_copy(z_hbm.at[idx_s[b]], rows_s[b], sem_r[b])
        pltpu.async_copy(z_hbm.at[idx_d[b]], rows_d[b], sem_r[b])

    def wait_rows(b):
        pltpu.make_async_copy(z_hbm.at[idx_s[b]], rows_s[b], sem_r[b]).wait()
        pltpu.make_async_copy(z_hbm.at[idx_d[b]], rows_d[b], sem_r[b]).wait()

    def compute_and_store(c, b):
        rs, rd = rows_s[b], rows_d[b]

        def dot_term(r, e, j):
            packed = plsc.bitcast(r[e, pl.ds(j * L, L)], jnp.bfloat16)
            return plsc.unpack(packed, format=plsc.PackFormat.INTERLEAVED)

        def sub_body(sb, _):
            e0 = sb * L
            for i in range(L):
                e = e0 + i
                acc = None
                for j in range(DP // L):
                    s_lo, s_hi = dot_term(rs, e, j)
                    d_lo, d_hi = dot_term(rd, e, j)
                    t = s_lo * d_lo + s_hi * d_hi
                    acc = t if acc is None else acc + t
                plsc.store_scatter(t16,
                                   [lane, jnp.full((L,), i, jnp.int32)],
                                   acc)
            s = t16[0, pl.ds(0, L)]
            for r in range(1, L):
                s += t16[r, pl.ds(0, L)]
            out_v[pl.ds(e0, L)] = 1.0 / (1.0 + jnp.exp(-s))
            return ()

        lax.fori_loop(0, CH // L, sub_body, ())
        pltpu.sync_copy(out_v, out_hbm.at[pl.ds(base_of(c), CH)])

    issue_idx(0, 0)
    issue_idx(1, 1)
    issue_idx(2, 2)
    wait_idx(0)
    issue_rows(0)
    wait_idx(1)
    issue_rows(1)

    def tri_body(g, _):
        for b in range(NB):
            c = NB * g + b
            nb2 = (b + 2) % NB
            wait_rows(b)
            issue_idx(c + NB, b)
            wait_idx(nb2)
            issue_rows(nb2)
            compute_and_store(c, b)
        return ()

    lax.fori_loop(0, (NCHUNK - 3) // NB, tri_body, ())

    wait_rows(0)
    wait_idx(2)
    issue_rows(2)
    compute_and_store(NCHUNK - 3, 0)
    wait_rows(1)
    compute_and_store(NCHUNK - 2, 1)
    wait_rows(2)
    compute_and_store(NCHUNK - 1, 2)


def kernel(z, edge_index):
    e = edge_index.astype(jnp.int32)
    src, dst = e[0], e[1]
    n_edges = src.shape[0]
    z16 = z.astype(jnp.bfloat16).reshape(z.shape[0], DP, 2)
    zp = jax.lax.bitcast_convert_type(z16, jnp.float32)
    assert n_edges % NW == 0
    assert (n_edges // NW - CH) % 8 == 0
    assert (n_edges // NW) <= NCHUNK * CH
    assert (NCHUNK - 3) % NB == 0
    run = functools.partial(
        pl.kernel,
        out_type=jax.ShapeDtypeStruct((n_edges,), jnp.float32),
        mesh=plsc.VectorSubcoreMesh(core_axis_name="c", subcore_axis_name="s"),
        compiler_params=pltpu.CompilerParams(needs_layout_passes=False,
                                             use_tc_tiling_on_sc=False),
        scratch_types=(
            [pltpu.VMEM((CH,), jnp.int32)] * 6
            + [pltpu.VMEM((CH, DP), jnp.float32)] * 6
            + [
                pltpu.VMEM((CH,), jnp.float32),
                pltpu.VMEM((L, L), jnp.float32),
            ]
            + [pltpu.SemaphoreType.DMA] * 6
        ),
    )(_decoder)
    return run(zp, src, dst)
